# Initial kernel scaffold; baseline (speedup 1.0000x reference)
import jax
import jax.numpy as jnp
from jax.experimental import pallas as pl


def kernel(edge_index, feat, feat_, W1, b1, gn1_w, gn1_b, gn1_ms, lin1_W, lin1_b, W2, b2, gn2_w, gn2_b, gn2_ms, lin2_W, lin2_b):
    raise NotImplementedError("write your pallas kernel here")


# speedup vs baseline: 11.9672x; 11.9672x over previous
import jax
import jax.numpy as jnp
from jax import lax
from jax.experimental import pallas as pl
from jax.experimental.pallas import tpu as pltpu
from jax.experimental.pallas import tpu_sc as plsc

N = 10000
E = 320000
D = 128
OUTD = 16
NLAYER = 4
NT = 16
NSC = 2
CH = 125
ROWS_E = E // CH
PT = ROWS_E // NT
RB = 624
EPS = 1e-5

_sc_mesh = plsc.VectorSubcoreMesh(core_axis_name="c", subcore_axis_name="s")


def _deg_body(src_r, ones_h, z1k, out_deg, didx, ones_v, stage_v, deg_sh, sem):
    c = lax.axis_index("c")
    s = lax.axis_index("s")

    @pl.when(s < 10)
    def _zero():
        pltpu.sync_copy(z1k, stage_v)
        pltpu.sync_copy(stage_v, deg_sh.at[pl.ds(s * 1000, 1000)])

    pltpu.sync_copy(ones_h, ones_v)
    rows_half = ROWS_E // NSC
    per_tile = rows_half // NT
    base = c * rows_half + s * per_tile
    pltpu.sync_copy(src_r.at[pl.ds(base, per_tile), :], didx)
    plsc.subcore_barrier()

    G = 5

    def outer(i, carry):
        for j in range(G):
            pltpu.async_copy(ones_v, deg_sh.at[didx.at[i * G + j]], sem,
                             add=True)
        for j in range(G):
            pltpu.make_async_copy(ones_v, deg_sh.at[didx.at[0]], sem).wait()
        return carry

    lax.fori_loop(0, per_tile // G, outer, 0)
    plsc.subcore_barrier()

    @pl.when(s < 10)
    def _readback():
        pltpu.sync_copy(deg_sh.at[pl.ds(s * 1000, 1000)], stage_v)
        pltpu.sync_copy(stage_v, out_deg.at[pl.ds(c * N + s * 1000, 1000)])


_deg_call = pl.kernel(
    _deg_body,
    out_type=jax.ShapeDtypeStruct((NSC * N,), jnp.float32),
    mesh=_sc_mesh,
    scratch_types=[
        pltpu.VMEM((ROWS_E // NSC // NT, CH), jnp.int32),
        pltpu.VMEM((CH,), jnp.float32),
        pltpu.VMEM((1000,), jnp.float32),
        pltpu.VMEM_SHARED((N,), jnp.float32),
        pltpu.SemaphoreType.DMA,
    ],
)


def _agg_body(idx_cat, zeros_h, u2d, out_a,
              sd0, sd1, rows0, rows1, a_sh, sg0, sg1, sx0, sx1):
    c = lax.axis_index("c")
    s = lax.axis_index("s")

    rbase = s * RB
    pltpu.sync_copy(zeros_h, a_sh.at[pl.ds(rbase, RB), :])

    @pl.when(s == NT - 1)
    def _zero_tail():
        pltpu.sync_copy(zeros_h.at[pl.ds(0, 16), :],
                        a_sh.at[pl.ds(NT * RB, 16), :])

    plsc.subcore_barrier()

    cb = s * PT
    last = ROWS_E - 1

    def step(k, sd_c, rows_c, sg_c, sx_c, sd_o, rows_o, sg_o, sx_o):
        pltpu.make_async_copy(idx_cat.at[c, 0], sd_o, sx_o).wait()
        pltpu.async_copy(u2d.at[sd_o.at[0]], rows_o, sg_o)
        pltpu.make_async_copy(u2d.at[sd_c.at[0]], rows_c, sg_c).wait()
        pltpu.sync_copy(rows_c, a_sh.at[sd_c.at[1]], add=True)
        nxt = jnp.minimum(cb + k + 2, last)
        pltpu.async_copy(idx_cat.at[c, nxt], sd_c, sx_c)

    pltpu.sync_copy(idx_cat.at[c, cb], sd0)
    pltpu.async_copy(u2d.at[sd0.at[0]], rows0, sg0)
    pltpu.async_copy(idx_cat.at[c, cb + 1], sd1, sx1)

    def outer(i, carry):
        k = i * 2
        step(k, sd0, rows0, sg0, sx0, sd1, rows1, sg1, sx1)
        step(k + 1, sd1, rows1, sg1, sx1, sd0, rows0, sg0, sx0)
        return carry

    lax.fori_loop(0, PT // 2, outer, 0)
    pltpu.make_async_copy(u2d.at[sd0.at[0]], rows0, sg0).wait()
    pltpu.make_async_copy(idx_cat.at[c, 0], sd1, sx1).wait()
    plsc.subcore_barrier()

    pltpu.sync_copy(a_sh.at[pl.ds(rbase, RB), :],
                    out_a.at[c, pl.ds(rbase, RB), :])

    @pl.when(s == NT - 1)
    def _read_tail():
        pltpu.sync_copy(a_sh.at[pl.ds(NT * RB, 16), :],
                        out_a.at[c, pl.ds(NT * RB, 16), :])


_agg_call = pl.kernel(
    _agg_body,
    out_type=jax.ShapeDtypeStruct((NSC, N, D), jnp.float32),
    mesh=_sc_mesh,
    scratch_types=[
        pltpu.VMEM((2, CH), jnp.int32),
        pltpu.VMEM((2, CH), jnp.int32),
        pltpu.VMEM((CH, D), jnp.float32),
        pltpu.VMEM((CH, D), jnp.float32),
        pltpu.VMEM_SHARED((N, D), jnp.float32),
        pltpu.SemaphoreType.DMA,
        pltpu.SemaphoreType.DMA,
        pltpu.SemaphoreType.DMA,
        pltpu.SemaphoreType.DMA,
    ],
)


def _prep_body(degT, feat, feat_, dis_o, u0_o):
    deg = degT[:, 0:1] + degT[:, 1:2]
    dis = jnp.where(deg > 0, lax.rsqrt(jnp.maximum(deg, 1.0)), 0.0)
    dis_o[...] = dis
    u0_o[0] = dis * feat[...]
    u0_o[1] = dis * feat_[...]


_prep_call = pl.pallas_call(
    _prep_body,
    out_shape=[
        jax.ShapeDtypeStruct((N, 1), jnp.float32),
        jax.ShapeDtypeStruct((NSC, N, D), jnp.float32),
    ],
)


def _cA_body(a0, dis_r, tx1_o, u1_o):
    dis = dis_r[...]
    tx1 = -dis * a0[...]
    tx1_o[...] = tx1
    u1_o[...] = dis * tx1


_cA_call = pl.pallas_call(
    _cA_body,
    out_shape=[
        jax.ShapeDtypeStruct((NSC, N, D), jnp.float32),
        jax.ShapeDtypeStruct((NSC, N, D), jnp.float32),
    ],
)


def _cB_body(a1, dis_r, x0, tx2_o, u2_o):
    dis = dis_r[...]
    tx2 = (-2.0) * dis * a1[...] - x0[...]
    tx2_o[...] = tx2
    u2_o[...] = dis * tx2


_cB_call = pl.pallas_call(
    _cB_body,
    out_shape=[
        jax.ShapeDtypeStruct((NSC, N, D), jnp.float32),
        jax.ShapeDtypeStruct((NSC, N, D), jnp.float32),
    ],
)


def _cheb_out(a2, dis, x0v, tx1v, tx2v, w, bvec, gw, gb, gms):
    tx3 = (-2.0) * dis * a2 - tx1v
    out = jnp.dot(x0v, w[0], preferred_element_type=jnp.float32)
    out = out + jnp.dot(tx1v, w[1], preferred_element_type=jnp.float32)
    out = out + jnp.dot(tx2v, w[2], preferred_element_type=jnp.float32)
    out = out + jnp.dot(tx3, w[3], preferred_element_type=jnp.float32)
    out = out + bvec
    mean = jnp.mean(out, axis=0, keepdims=True)
    o = out - mean * gms
    var = jnp.mean(o * o, axis=0, keepdims=True)
    return gw * o / jnp.sqrt(var + EPS) + gb


def _le_body(a2, dis_r, x0, tx1, tx2, W, b, gw, gb, gms, xn_o, un_o):
    dis = dis_r[...]
    g = _cheb_out(a2[...], dis, x0[...], tx1[...], tx2[...], W[...],
                  b[...], gw[...], gb[...], gms[...])
    x = jnp.where(g >= 0, g, 0.1 * g)
    xn_o[...] = x
    un_o[...] = dis * x


_le_call = pl.pallas_call(
    _le_body,
    out_shape=[
        jax.ShapeDtypeStruct((N, D), jnp.float32),
        jax.ShapeDtypeStruct((N, D), jnp.float32),
    ],
)


def _fin_body(a2, dis_r, x0, tx1, tx2, W, b, gw, gb, gms,
              featS, linW, linb, res_o):
    dis = dis_r[...]
    g = _cheb_out(a2[...], dis, x0[...], tx1[...], tx2[...], W[...],
                  b[...], gw[...], gb[...], gms[...])
    f = jnp.maximum(featS[...] + g, 0.0)
    pool = jnp.maximum(jnp.mean(f, axis=0, keepdims=True), 0.0)
    o = jnp.dot(pool, linW[...], preferred_element_type=jnp.float32) + linb[...]
    m = jnp.max(o, axis=-1, keepdims=True)
    ex = jnp.exp(o - m)
    sm = ex / jnp.sum(ex, axis=-1, keepdims=True)
    res_o[...] = sm * jnp.maximum(o, 0.0)


_fin_call = pl.pallas_call(
    _fin_body,
    out_shape=[jax.ShapeDtypeStruct((1, OUTD), jnp.float32)],
)


def kernel(edge_index, feat, feat_, W1, b1, gn1_w, gn1_b, gn1_ms, lin1_W,
           lin1_b, W2, b2, gn2_w, gn2_b, gn2_ms, lin2_W, lin2_b):
    src_r = edge_index[0].reshape(ROWS_E, CH)
    dst_r = edge_index[1].reshape(ROWS_E, CH)
    idx_cat = jnp.stack([jnp.stack([src_r, dst_r], axis=1),
                         jnp.stack([src_r + N, dst_r], axis=1)])
    ones_h = jnp.ones((CH,), jnp.float32)
    z1k = jnp.zeros((1000,), jnp.float32)
    zeros_h = jnp.zeros((RB, D), jnp.float32)

    deg2 = _deg_call(src_r, ones_h, z1k)
    degT = deg2.reshape(NSC, N).T
    dis, u = _prep_call(degT, feat, feat_)

    x = jnp.stack([feat, feat_])
    featS = x
    Wb = jnp.stack([W1, W2])
    bb = jnp.stack([b1, b2])
    gwb = jnp.stack([gn1_w, gn2_w])
    gbb = jnp.stack([gn1_b, gn2_b])
    gmsb = jnp.stack([gn1_ms, gn2_ms])
    linW = jnp.stack([lin1_W, lin2_W])
    linb = jnp.stack([lin1_b, lin2_b]).reshape(NSC, 1, OUTD)

    for l in range(NLAYER):
        a0 = _agg_call(idx_cat, zeros_h, u.reshape(NSC * N, D))
        tx1, u1 = _cA_call(a0, dis)
        a1 = _agg_call(idx_cat, zeros_h, u1.reshape(NSC * N, D))
        tx2, u2 = _cB_call(a1, dis, x)
        a2 = _agg_call(idx_cat, zeros_h, u2.reshape(NSC * N, D))
        Wl = Wb[:, l]
        bl = bb[:, l].reshape(NSC, 1, D)
        gwl = gwb[:, l].reshape(NSC, 1, D)
        gbl = gbb[:, l].reshape(NSC, 1, D)
        gmsl = gmsb[:, l].reshape(NSC, 1, D)
        if l < NLAYER - 1:
            xs, us = [], []
            for b in range(NSC):
                xb, ub = _le_call(a2[b], dis, x[b], tx1[b], tx2[b], Wl[b],
                                  bl[b], gwl[b], gbl[b], gmsl[b])
                xs.append(xb)
                us.append(ub)
            x = jnp.stack(xs)
            u = jnp.stack(us)
        else:
            res = []
            for b in range(NSC):
                (rb,) = _fin_call(a2[b], dis, x[b], tx1[b], tx2[b], Wl[b],
                                  bl[b], gwl[b], gbl[b], gmsl[b],
                                  featS[b], linW[b], linb[b])
                res.append(rb)
    return (res[0][0], res[1][0])


# speedup vs baseline: 13.5405x; 1.1315x over previous
import jax
import jax.numpy as jnp
from jax import lax
from jax.experimental import pallas as pl
from jax.experimental.pallas import tpu as pltpu
from jax.experimental.pallas import tpu_sc as plsc

N = 10000
E = 320000
D = 128
OUTD = 16
NLAYER = 4
NT = 16
NSC = 2
CH = 125
ROWS_E = E // CH
PT = ROWS_E // NT
RB = 624
EPS = 1e-5

_sc_mesh = plsc.VectorSubcoreMesh(core_axis_name="c", subcore_axis_name="s")


def _deg_body(src_r, ones_h, z1k, out_deg, didx, ones_v, stage_v, deg_sh, sem):
    c = lax.axis_index("c")
    s = lax.axis_index("s")

    @pl.when(s < 10)
    def _zero():
        pltpu.sync_copy(z1k, stage_v)
        pltpu.sync_copy(stage_v, deg_sh.at[pl.ds(s * 1000, 1000)])

    pltpu.sync_copy(ones_h, ones_v)
    rows_half = ROWS_E // NSC
    per_tile = rows_half // NT
    base = c * rows_half + s * per_tile
    pltpu.sync_copy(src_r.at[pl.ds(base, per_tile), :], didx)
    plsc.subcore_barrier()

    G = 5

    def outer(i, carry):
        for j in range(G):
            pltpu.async_copy(ones_v, deg_sh.at[didx.at[i * G + j]], sem,
                             add=True)
        for j in range(G):
            pltpu.make_async_copy(ones_v, deg_sh.at[didx.at[0]], sem).wait()
        return carry

    lax.fori_loop(0, per_tile // G, outer, 0)
    plsc.subcore_barrier()

    @pl.when(s < 10)
    def _readback():
        pltpu.sync_copy(deg_sh.at[pl.ds(s * 1000, 1000)], stage_v)
        pltpu.sync_copy(stage_v, out_deg.at[pl.ds(c * N + s * 1000, 1000)])


_deg_call = pl.kernel(
    _deg_body,
    out_type=jax.ShapeDtypeStruct((NSC * N,), jnp.float32),
    mesh=_sc_mesh,
    scratch_types=[
        pltpu.VMEM((ROWS_E // NSC // NT, CH), jnp.int32),
        pltpu.VMEM((CH,), jnp.float32),
        pltpu.VMEM((1000,), jnp.float32),
        pltpu.VMEM_SHARED((N,), jnp.float32),
        pltpu.SemaphoreType.DMA,
    ],
)


def _agg_body(idx_cat, zeros_h, u2d, out_a,
              sd0, sd1, rows0, rows1, a_sh, sg0, sg1, sx0, sx1, ss0, ss1):
    c = lax.axis_index("c")
    s = lax.axis_index("s")

    rbase = s * RB
    pltpu.sync_copy(zeros_h, a_sh.at[pl.ds(rbase, RB), :])

    @pl.when(s == NT - 1)
    def _zero_tail():
        pltpu.sync_copy(zeros_h.at[pl.ds(0, 16), :],
                        a_sh.at[pl.ds(NT * RB, 16), :])

    cb = s * PT
    bidx = (sd0, sd1)
    rows = (rows0, rows1)
    sx = (sx0, sx1)
    sg = (sg0, sg1)
    ss = (ss0, ss1)

    pltpu.async_copy(idx_cat.at[c, pl.ds(cb, 8)], bidx[0], sx[0])
    plsc.subcore_barrier()

    def do_block(b_row, sc, first_block=False):
        so = 1 - sc
        for j in range(8):
            r = j % 2
            rr = 1 - r
            if j == 0:
                pltpu.make_async_copy(idx_cat.at[c, pl.ds(0, 8)],
                                      bidx[sc], sx[sc]).wait()
            if not (first_block and j < 2):
                pltpu.make_async_copy(rows[r], a_sh.at[bidx[sc].at[0, 1]],
                                      ss[r]).wait()
            pltpu.async_copy(u2d.at[bidx[sc].at[j, 0, pl.ds(0, 64)]],
                             rows[r].at[pl.ds(0, 64), :], sg[r])
            pltpu.async_copy(u2d.at[bidx[sc].at[j, 0, pl.ds(64, 61)]],
                             rows[r].at[pl.ds(64, 61), :], sg[r])
            if not (first_block and j == 0):
                didx_ref = bidx[so].at[7, 1] if j == 0 else bidx[sc].at[j - 1, 1]
                pltpu.make_async_copy(u2d.at[bidx[sc].at[j, 0]],
                                      rows[rr], sg[rr]).wait()
                pltpu.async_copy(rows[rr], a_sh.at[didx_ref], ss[rr], add=True)
            if j == 2:
                nrow = jnp.minimum(b_row + 8, ROWS_E - 8)
                pltpu.async_copy(idx_cat.at[c, pl.ds(nrow, 8)],
                                 bidx[so], sx[so])

    do_block(cb, 0, first_block=True)
    do_block(cb + 8, 1)

    def outer(i, carry):
        b_row = cb + 16 + i * 16
        do_block(b_row, 0)
        do_block(b_row + 8, 1)
        return carry

    lax.fori_loop(0, (PT - 16) // 16, outer, 0)

    pltpu.make_async_copy(u2d.at[bidx[1].at[7, 0]], rows1, sg1).wait()
    pltpu.async_copy(rows1, a_sh.at[bidx[1].at[7, 1]], ss1, add=True)
    pltpu.make_async_copy(rows0, a_sh.at[bidx[1].at[0, 1]], ss0).wait()
    pltpu.make_async_copy(rows1, a_sh.at[bidx[1].at[0, 1]], ss1).wait()
    pltpu.make_async_copy(idx_cat.at[c, pl.ds(0, 8)], bidx[0], sx[0]).wait()
    plsc.subcore_barrier()

    pltpu.sync_copy(a_sh.at[pl.ds(rbase, RB), :],
                    out_a.at[c, pl.ds(rbase, RB), :])

    @pl.when(s == NT - 1)
    def _read_tail():
        pltpu.sync_copy(a_sh.at[pl.ds(NT * RB, 16), :],
                        out_a.at[c, pl.ds(NT * RB, 16), :])


_agg_call = pl.kernel(
    _agg_body,
    out_type=jax.ShapeDtypeStruct((NSC, N, D), jnp.float32),
    mesh=_sc_mesh,
    scratch_types=[
        pltpu.VMEM((8, 2, CH), jnp.int32),
        pltpu.VMEM((8, 2, CH), jnp.int32),
        pltpu.VMEM((CH, D), jnp.float32),
        pltpu.VMEM((CH, D), jnp.float32),
        pltpu.VMEM_SHARED((N, D), jnp.float32),
        pltpu.SemaphoreType.DMA,
        pltpu.SemaphoreType.DMA,
        pltpu.SemaphoreType.DMA,
        pltpu.SemaphoreType.DMA,
        pltpu.SemaphoreType.DMA,
        pltpu.SemaphoreType.DMA,
    ],
)


def _prep_body(degT, feat, feat_, dis_o, u0_o):
    deg = degT[:, 0:1] + degT[:, 1:2]
    dis = jnp.where(deg > 0, lax.rsqrt(jnp.maximum(deg, 1.0)), 0.0)
    dis_o[...] = dis
    u0_o[0] = dis * feat[...]
    u0_o[1] = dis * feat_[...]


_prep_call = pl.pallas_call(
    _prep_body,
    out_shape=[
        jax.ShapeDtypeStruct((N, 1), jnp.float32),
        jax.ShapeDtypeStruct((NSC, N, D), jnp.float32),
    ],
)


def _cA_body(a0, dis_r, tx1_o, u1_o):
    dis = dis_r[...]
    tx1 = -dis * a0[...]
    tx1_o[...] = tx1
    u1_o[...] = dis * tx1


_cA_call = pl.pallas_call(
    _cA_body,
    out_shape=[
        jax.ShapeDtypeStruct((NSC, N, D), jnp.float32),
        jax.ShapeDtypeStruct((NSC, N, D), jnp.float32),
    ],
)


def _cB_body(a1, dis_r, x0, tx2_o, u2_o):
    dis = dis_r[...]
    tx2 = (-2.0) * dis * a1[...] - x0[...]
    tx2_o[...] = tx2
    u2_o[...] = dis * tx2


_cB_call = pl.pallas_call(
    _cB_body,
    out_shape=[
        jax.ShapeDtypeStruct((NSC, N, D), jnp.float32),
        jax.ShapeDtypeStruct((NSC, N, D), jnp.float32),
    ],
)


def _cheb_out(a2, dis, x0v, tx1v, tx2v, w, bvec, gw, gb, gms):
    tx3 = (-2.0) * dis * a2 - tx1v
    out = jnp.dot(x0v, w[0], preferred_element_type=jnp.float32)
    out = out + jnp.dot(tx1v, w[1], preferred_element_type=jnp.float32)
    out = out + jnp.dot(tx2v, w[2], preferred_element_type=jnp.float32)
    out = out + jnp.dot(tx3, w[3], preferred_element_type=jnp.float32)
    out = out + bvec
    mean = jnp.mean(out, axis=0, keepdims=True)
    o = out - mean * gms
    var = jnp.mean(o * o, axis=0, keepdims=True)
    return gw * o / jnp.sqrt(var + EPS) + gb


def _le_body(a2, dis_r, x0, tx1, tx2, W, b, gw, gb, gms, xn_o, un_o):
    dis = dis_r[...]
    g = _cheb_out(a2[...], dis, x0[...], tx1[...], tx2[...], W[...],
                  b[...], gw[...], gb[...], gms[...])
    x = jnp.where(g >= 0, g, 0.1 * g)
    xn_o[...] = x
    un_o[...] = dis * x


_le_call = pl.pallas_call(
    _le_body,
    out_shape=[
        jax.ShapeDtypeStruct((N, D), jnp.float32),
        jax.ShapeDtypeStruct((N, D), jnp.float32),
    ],
)


def _fin_body(a2, dis_r, x0, tx1, tx2, W, b, gw, gb, gms,
              featS, linW, linb, res_o):
    dis = dis_r[...]
    g = _cheb_out(a2[...], dis, x0[...], tx1[...], tx2[...], W[...],
                  b[...], gw[...], gb[...], gms[...])
    f = jnp.maximum(featS[...] + g, 0.0)
    pool = jnp.maximum(jnp.mean(f, axis=0, keepdims=True), 0.0)
    o = jnp.dot(pool, linW[...], preferred_element_type=jnp.float32) + linb[...]
    m = jnp.max(o, axis=-1, keepdims=True)
    ex = jnp.exp(o - m)
    sm = ex / jnp.sum(ex, axis=-1, keepdims=True)
    res_o[...] = sm * jnp.maximum(o, 0.0)


_fin_call = pl.pallas_call(
    _fin_body,
    out_shape=[jax.ShapeDtypeStruct((1, OUTD), jnp.float32)],
)


def kernel(edge_index, feat, feat_, W1, b1, gn1_w, gn1_b, gn1_ms, lin1_W,
           lin1_b, W2, b2, gn2_w, gn2_b, gn2_ms, lin2_W, lin2_b):
    src_r = edge_index[0].reshape(ROWS_E, CH)
    dst_r = edge_index[1].reshape(ROWS_E, CH)
    idx_cat = jnp.stack([jnp.stack([src_r, dst_r], axis=1),
                         jnp.stack([src_r + N, dst_r], axis=1)])
    ones_h = jnp.ones((CH,), jnp.float32)
    z1k = jnp.zeros((1000,), jnp.float32)
    zeros_h = jnp.zeros((RB, D), jnp.float32)

    deg2 = _deg_call(src_r, ones_h, z1k)
    degT = deg2.reshape(NSC, N).T
    dis, u = _prep_call(degT, feat, feat_)

    x = jnp.stack([feat, feat_])
    featS = x
    Wb = jnp.stack([W1, W2])
    bb = jnp.stack([b1, b2])
    gwb = jnp.stack([gn1_w, gn2_w])
    gbb = jnp.stack([gn1_b, gn2_b])
    gmsb = jnp.stack([gn1_ms, gn2_ms])
    linW = jnp.stack([lin1_W, lin2_W])
    linb = jnp.stack([lin1_b, lin2_b]).reshape(NSC, 1, OUTD)

    for l in range(NLAYER):
        a0 = _agg_call(idx_cat, zeros_h, u.reshape(NSC * N, D))
        tx1, u1 = _cA_call(a0, dis)
        a1 = _agg_call(idx_cat, zeros_h, u1.reshape(NSC * N, D))
        tx2, u2 = _cB_call(a1, dis, x)
        a2 = _agg_call(idx_cat, zeros_h, u2.reshape(NSC * N, D))
        Wl = Wb[:, l]
        bl = bb[:, l].reshape(NSC, 1, D)
        gwl = gwb[:, l].reshape(NSC, 1, D)
        gbl = gbb[:, l].reshape(NSC, 1, D)
        gmsl = gmsb[:, l].reshape(NSC, 1, D)
        if l < NLAYER - 1:
            xs, us = [], []
            for b in range(NSC):
                xb, ub = _le_call(a2[b], dis, x[b], tx1[b], tx2[b], Wl[b],
                                  bl[b], gwl[b], gbl[b], gmsl[b])
                xs.append(xb)
                us.append(ub)
            x = jnp.stack(xs)
            u = jnp.stack(us)
        else:
            res = []
            for b in range(NSC):
                (rb,) = _fin_call(a2[b], dis, x[b], tx1[b], tx2[b], Wl[b],
                                  bl[b], gwl[b], gbl[b], gmsl[b],
                                  featS[b], linW[b], linb[b])
                res.append(rb)
    return (res[0][0], res[1][0])
